# baseline (speedup 1.0000x reference)
import jax
import jax.numpy as jnp
from jax import lax
from jax.experimental import pallas as pl
from jax.experimental.pallas import tpu as pltpu
from jax.experimental.pallas import tpu_sc as plsc

EMBED = 64
BATCH = 16384
NROWS = 100000
NC = 2
NS = 16
L = 16
NW = NC * NS
BPW = BATCH // NW
W = 512
LASTW = NROWS - 195 * W
HCAP = 1616
CCAP = 208
GCAP = CCAP // L
SENT = BATCH
SROWS = BATCH + 8
SW = 128
GWORDS = L * SW


def _extract_side(tblT, idx_hbm, stage_hbm, uvals, hits, ccols, cdest,
                  bufs, tailbuf, stags, tmp, cidx, drainbuf, insems, outsems):
    s = lax.axis_index("s")
    lanes = lax.iota(jnp.int32, L)
    rows_e = [e * L + lanes for e in range(EMBED // L)]

    def blk_off(k):
        return (s + 16 * k) * W

    for b in range(2):
        pltpu.async_copy(tblT.at[:, pl.ds(blk_off(b), W)], bufs[b], insems[b])

    pltpu.sync_copy(idx_hbm, uvals)

    def filt(i, cnt):
        uv = uvals[pl.ds(i * L, L)]
        m = ((uv >> 9) & 15) == s
        key = ((uv >> 13) << 23) | ((uv & 511) << 14) | (i * L + lanes)
        plsc.store_compressed(hits.at[pl.ds(cnt, L)], key, mask=m)
        return cnt + plsc.all_reduce_population_count(m)[0]

    cnt = lax.fori_loop(0, BATCH // L, filt, jnp.int32(0))
    full_mask = lanes >= 0
    plsc.store_compressed(hits.at[pl.ds(cnt, L)],
                          jnp.full((L,), 15 << 23, jnp.int32), mask=full_mask)
    nhv = (cnt + L - 1) >> 4

    def drain_one(b):
        pltpu.make_async_copy(
            stage_hbm.at[pl.ds(0, L)], drainbuf, outsems[b]).wait()

    def process_block(kidx, b, buf, colhi=None, colshift=0, gate=None):
        def subf(i, cc):
            key = hits[pl.ds(i * L, L)]
            m = (key >> 23) == kidx
            col = (key >> 14) & 511
            if colhi is not None:
                m = m & (col >= colshift) & (col < colhi)
            if gate is not None:
                m = m & gate
            plsc.store_compressed(
                ccols.at[pl.ds(cc, L)], col - colshift, mask=m)
            plsc.store_compressed(
                cdest.at[pl.ds(cc, L)], key & 16383, mask=m)
            return cc + plsc.all_reduce_population_count(m)[0]

        cc = lax.fori_loop(0, nhv, subf, jnp.int32(0))
        plsc.store_compressed(ccols.at[pl.ds(cc, L)],
                              jnp.zeros((L,), jnp.int32), mask=full_mask)
        plsc.store_compressed(cdest.at[pl.ds(cc, L)],
                              jnp.full((L,), SENT, jnp.int32), mask=full_mask)

        def group(g, carry):
            colv = ccols[pl.ds(g * L, L)]
            destv = cdest[pl.ds(g * L, L)]
            for e in range(EMBED):
                row = jnp.full((L,), e, jnp.int32)
                vals = plsc.load_gather(buf, [row, colv])
                plsc.store_compressed(tmp.at[pl.ds(e * 17, L)], vals,
                                      mask=full_mask)
            slot = g & 3

            @pl.when(g >= 4)
            def _():
                drain_one(b)
            for l in range(L):
                for e in range(EMBED // L):
                    stags[b][slot, l, pl.ds(e * L, L)] = plsc.load_gather(
                        tmp, [(rows_e[e]) * 17 + l])
            cidx[slot, 0, pl.ds(0, L)] = destv
            pltpu.async_copy(stags[b].at[slot], stage_hbm.at[cidx.at[slot, 0]],
                             outsems[b])
            return carry

        ngroups = (cc + L - 1) >> 4
        lax.fori_loop(0, ngroups, group, 0)

        def fin(g, carry):
            drain_one(b)
            return carry
        lax.fori_loop(0, jnp.minimum(ngroups, 4), fin, 0)
        return ngroups

    def pair(p, carry):
        for b in range(2):
            k = 2 * p + b
            pltpu.make_async_copy(
                tblT.at[:, pl.ds(blk_off(k), W)], bufs[b], insems[b]).wait()
            process_block(k, b, bufs[b])

            @pl.when(k <= 9)
            def _():
                pltpu.async_copy(tblT.at[:, pl.ds(blk_off(k + 2), W)],
                                 bufs[b], insems[b])
        return carry

    lax.fori_loop(0, 6, pair, 0)

    @pl.when(s <= 2)
    def _():
        pltpu.async_copy(
            tblT.at[:, pl.ds(blk_off(12), W)], bufs[0], insems[0])
        pltpu.make_async_copy(
            tblT.at[:, pl.ds(blk_off(12), W)], bufs[0], insems[0]).wait()

    @pl.when(s == 3)
    def _():
        pltpu.async_copy(tblT.at[:, pl.ds(195 * W, 128)],
                         bufs[0].at[:, pl.ds(0, 128)], insems[0])
        pltpu.async_copy(tblT.at[:, pl.ds(195 * W + 128, 32)],
                         tailbuf, insems[1])
        pltpu.make_async_copy(tblT.at[:, pl.ds(195 * W, 128)],
                              bufs[0].at[:, pl.ds(0, 128)], insems[0]).wait()
        pltpu.make_async_copy(tblT.at[:, pl.ds(195 * W + 128, 32)],
                              tailbuf, insems[1]).wait()

    is_s3 = lax.broadcast(s == 3, (L,))
    colhi_a = jnp.where(is_s3, 128, W)
    process_block(12, 0, bufs[0], colhi=colhi_a)
    process_block(12, 1, tailbuf, colhi=160, colshift=128, gate=is_s3)


def _extract_body(u_hbm, v_hbm, utT, itT, ustage, vstage, uvals, hits,
                  ccols, cdest, buf0, buf1, tailbuf, stag0, stag1, tmp,
                  cidx, drainbuf, insem0, insem1, outsem0, outsem1):
    c = lax.axis_index("c")

    @pl.when(c == 0)
    def _():
        _extract_side(utT, u_hbm, ustage, uvals, hits, ccols, cdest,
                      (buf0, buf1), tailbuf, (stag0, stag1), tmp, cidx, drainbuf,
                      (insem0, insem1), (outsem0, outsem1))

    @pl.when(c == 1)
    def _():
        _extract_side(itT, v_hbm, vstage, uvals, hits, ccols, cdest,
                      (buf0, buf1), tailbuf, (stag0, stag1), tmp, cidx, drainbuf,
                      (insem0, insem1), (outsem0, outsem1))


def _dot_body(ustage, vstage, out_hbm, ubuf, vbuf, outv, sem):
    wid = lax.axis_index("s") * NC + lax.axis_index("c")
    base = wid * BPW
    lanes = lax.iota(jnp.int32, L)
    HALF = BPW // 2

    for h in range(2):
        hb = base + h * HALF
        cu = pltpu.async_copy(ustage.at[pl.ds(hb, HALF)], ubuf, sem)
        cv = pltpu.async_copy(vstage.at[pl.ds(hb, HALF)], vbuf, sem)
        cu.wait()
        cv.wait()

        def group(g, carry):
            tot = jnp.zeros((L,), jnp.float32)
            for r in range(L):
                j = g * L + r
                acc = ubuf[j, pl.ds(0, L)] * vbuf[j, pl.ds(0, L)]
                for e in range(1, EMBED // L):
                    acc = acc + (ubuf[j, pl.ds(e * L, L)]
                                 * vbuf[j, pl.ds(e * L, L)])
                tot = jnp.where(lanes == r, jnp.sum(acc), tot)
            outv[pl.ds(h * HALF + g * L, L)] = tot
            return carry

        lax.fori_loop(0, HALF // L, group, 0)
    pltpu.sync_copy(outv, out_hbm.at[pl.ds(base, BPW)])


def kernel(u, v, user_table, item_table):
    u32 = u.astype(jnp.int32)
    v32 = v.astype(jnp.int32)
    utT = user_table.T
    itT = item_table.T
    mesh = plsc.VectorSubcoreMesh(core_axis_name="c", subcore_axis_name="s")
    params = pltpu.CompilerParams(
        needs_layout_passes=False, use_tc_tiling_on_sc=True)

    extract = pl.kernel(
        _extract_body,
        out_type=(
            jax.ShapeDtypeStruct((SROWS, SW), jnp.float32),
            jax.ShapeDtypeStruct((SROWS, SW), jnp.float32),
        ),
        mesh=mesh,
        compiler_params=params,
        scratch_types=[
            pltpu.VMEM((BATCH,), jnp.int32),
            pltpu.VMEM((HCAP,), jnp.int32),
            pltpu.VMEM((CCAP,), jnp.int32),
            pltpu.VMEM((CCAP,), jnp.int32),
            pltpu.VMEM((EMBED, W), jnp.float32),
            pltpu.VMEM((EMBED, W), jnp.float32),
            pltpu.VMEM((EMBED, 32), jnp.float32),
            pltpu.VMEM((4, L, SW), jnp.float32),
            pltpu.VMEM((4, L, SW), jnp.float32),
            pltpu.VMEM((EMBED * 17,), jnp.float32),
            pltpu.VMEM((4, 1, L), jnp.int32),
            pltpu.VMEM((L, SW), jnp.float32),
            pltpu.SemaphoreType.DMA,
            pltpu.SemaphoreType.DMA,
            pltpu.SemaphoreType.DMA,
            pltpu.SemaphoreType.DMA,
        ],
    )
    ustage, vstage = extract(u32, v32, utT, itT)

    dot = pl.kernel(
        _dot_body,
        out_type=jax.ShapeDtypeStruct((BATCH,), jnp.float32),
        mesh=mesh,
        compiler_params=params,
        scratch_types=[
            pltpu.VMEM((BPW // 2, SW), jnp.float32),
            pltpu.VMEM((BPW // 2, SW), jnp.float32),
            pltpu.VMEM((BPW,), jnp.float32),
            pltpu.SemaphoreType.DMA,
        ],
    )
    return dot(ustage, vstage)


# speedup vs baseline: 1.5773x; 1.5773x over previous
import jax
import jax.numpy as jnp
from jax import lax
from jax.experimental import pallas as pl
from jax.experimental.pallas import tpu as pltpu
from jax.experimental.pallas import tpu_sc as plsc

EMBED = 64
BATCH = 16384
NC = 2
NS = 16
L = 16
NW = NC * NS
BPW = BATCH // NW
C = 32
NCH = BPW // C


def _sc_body(u_hbm, v_hbm, ut_hbm, it_hbm, out_hbm,
             uidx, vidx, ubuf, vbuf, ubuf2, vbuf2, drain, outv, sem, sem2):
    wid = lax.axis_index("s") * NC + lax.axis_index("c")
    base = wid * BPW

    pltpu.sync_copy(u_hbm.at[wid], uidx)
    pltpu.sync_copy(v_hbm.at[wid], vidx)

    lanes = lax.iota(jnp.int32, L)

    def fire(ci, b):
        for g in range(C // L):
            uvec = uidx[pl.ds(ci * C + g * L, L)]
            vvec = vidx[pl.ds(ci * C + g * L, L)]
            for r in range(L):
                j = g * L + r
                pltpu.async_copy(ut_hbm.at[uvec[r]], ubufs[b].at[j], sems[b])
                pltpu.async_copy(it_hbm.at[vvec[r]], vbufs[b].at[j], sems[b])

    def compute(ci, b):
        for g in range(C // L):
            tot = jnp.zeros((L,), jnp.float32)
            for r in range(L):
                j = g * L + r
                acc = ubufs[b][j, pl.ds(0, L)] * vbufs[b][j, pl.ds(0, L)]
                for e in range(1, EMBED // L):
                    acc = acc + (ubufs[b][j, pl.ds(e * L, L)]
                                 * vbufs[b][j, pl.ds(e * L, L)])
                tot = jnp.where(lanes == r, jnp.sum(acc), tot)
            outv[pl.ds(ci * C + g * L, L)] = tot

    ubufs = (ubuf, ubuf2)
    vbufs = (vbuf, vbuf2)
    sems = (sem, sem2)

    for b in range(2):
        fire(b, b)

    def pairs(p, carry):
        for b in range(2):
            ci = 2 * p + b
            pltpu.make_async_copy(
                ut_hbm.at[pl.ds(0, C)], drain, sems[b]).wait()
            pltpu.make_async_copy(
                it_hbm.at[pl.ds(0, C)], drain, sems[b]).wait()

            compute(ci, b)

            @pl.when(ci < NCH - 2)
            def _():
                fire(ci + 2, b)
        return carry

    lax.fori_loop(0, NCH // 2, pairs, 0)
    pltpu.sync_copy(outv, out_hbm.at[pl.ds(base, BPW)])


def kernel(u, v, user_table, item_table):
    u2 = u.astype(jnp.int32).reshape(NW, BPW)
    v2 = v.astype(jnp.int32).reshape(NW, BPW)
    mesh = plsc.VectorSubcoreMesh(core_axis_name="c", subcore_axis_name="s")
    f = pl.kernel(
        _sc_body,
        out_type=jax.ShapeDtypeStruct((BATCH,), jnp.float32),
        mesh=mesh,
        compiler_params=pltpu.CompilerParams(
            needs_layout_passes=False, use_tc_tiling_on_sc=True),
        scratch_types=[
            pltpu.VMEM((BPW,), jnp.int32),
            pltpu.VMEM((BPW,), jnp.int32),
            pltpu.VMEM((C, EMBED), jnp.float32),
            pltpu.VMEM((C, EMBED), jnp.float32),
            pltpu.VMEM((C, EMBED), jnp.float32),
            pltpu.VMEM((C, EMBED), jnp.float32),
            pltpu.VMEM((C, EMBED), jnp.float32),
            pltpu.VMEM((BPW,), jnp.float32),
            pltpu.SemaphoreType.DMA,
            pltpu.SemaphoreType.DMA,
        ],
    )
    return f(u2, v2, user_table, item_table)
